# Initial kernel scaffold; baseline (speedup 1.0000x reference)
import jax
import jax.numpy as jnp
from jax.experimental import pallas as pl


def kernel(nodes, triples, weights, bias):
    raise NotImplementedError("write your pallas kernel here")


# speedup vs baseline: 60.4191x; 60.4191x over previous
import functools

import jax
import jax.numpy as jnp
from jax import lax
from jax.experimental import pallas as pl
from jax.experimental.pallas import tpu as pltpu
from jax.experimental.pallas import tpu_sc as plsc

N = 10000
R = 16
E = 320000
H0 = 128
H1 = 16

NC = 2
NS = 16
L = 16

SUB = 80
EROWS = E // SUB
RPB = 25
BLKE = RPB * SUB
AROWS = EROWS // NS
ABLK = AROWS // RPB
BROWS = EROWS // (NC * NS)
BBLK = BROWS // RPB
CSL = (R * N) // NS
OSL = N // NS

_GDN = lax.GatherDimensionNumbers(
    offset_dims=(), collapsed_slice_dims=(0,), start_index_map=(0,))


def _mm_body(n_ref, w_ref, o_ref):
    o_ref[...] = jnp.dot(n_ref[...], w_ref[...],
                         preferred_element_type=jnp.float32)


_BM = 2000
_mm = pl.pallas_call(
    _mm_body,
    grid=(N // _BM,),
    in_specs=[
        pl.BlockSpec((_BM, H0), lambda i: (i, 0)),
        pl.BlockSpec((H0, R * H1), lambda i: (0, 0)),
    ],
    out_specs=pl.BlockSpec((_BM, R * H1), lambda i: (i, 0)),
    out_shape=jax.ShapeDtypeStruct((N, R * H1), jnp.float32),
)


def _comb_body(p_ref, b_ref, o_ref):
    o_ref[...] = p_ref[0] + p_ref[1] + b_ref[...]


_comb = pl.pallas_call(
    _comb_body,
    in_specs=[
        pl.BlockSpec((NC, (N * H1) // 128, 128), lambda: (0, 0, 0)),
        pl.BlockSpec((1, 128), lambda: (0, 0)),
    ],
    out_specs=pl.BlockSpec(((N * H1) // 128, 128), lambda: (0, 0)),
    out_shape=jax.ShapeDtypeStruct(((N * H1) // 128, 128), jnp.float32),
)


def _sc_body(s2, p2, o2, nw2, part,
             counts_sp, out_sp, pb, qb, sb, keyb, valb, ones, rows, sem):
    cid = lax.axis_index("c")
    sid = lax.axis_index("s")

    for k in range(SUB // L):
        ones[pl.ds(k * L, L)] = jnp.ones((L,), jnp.float32)

    def zf(i, _):
        valb[pl.ds(i * L, L)] = jnp.zeros((L,), jnp.float32)
        return 0

    lax.fori_loop(0, BLKE // L, zf, 0)

    def zr(i, _):
        rows[i] = jnp.zeros((L,), jnp.float32)
        return 0

    lax.fori_loop(0, OSL, zr, 0)

    for t in range(CSL // BLKE):
        pltpu.sync_copy(valb, counts_sp.at[pl.ds(sid * CSL + t * BLKE, BLKE)])
    pltpu.sync_copy(rows.at[pl.ds(0, OSL)], out_sp.at[pl.ds(sid * OSL, OSL)])
    plsc.subcore_barrier()

    arow0 = sid * AROWS

    def pa(b, _):
        r0 = arow0 + b * RPB
        pltpu.sync_copy(p2.at[pl.ds(r0, RPB)], pb)
        pltpu.sync_copy(s2.at[pl.ds(r0, RPB)], sb)

        def mk(j, _):
            for k in range(SUB // L):
                sl = pl.ds(k * L, L)
                keyb[j, sl] = pb[j, sl] * N + sb[j, sl]
            return 0

        lax.fori_loop(0, RPB, mk, 0)

        def scat(j, _):
            pltpu.sync_copy(ones, counts_sp.at[keyb.at[j]], add=True)
            return 0

        lax.fori_loop(0, RPB, scat, 0)
        return 0

    lax.fori_loop(0, ABLK, pa, 0)
    plsc.subcore_barrier()

    brow0 = cid * (EROWS // NC) + sid * BROWS

    def pb_loop(g, _):
        r0 = brow0 + g * RPB
        pltpu.sync_copy(p2.at[pl.ds(r0, RPB)], pb)
        pltpu.sync_copy(o2.at[pl.ds(r0, RPB)], qb)
        pltpu.sync_copy(s2.at[pl.ds(r0, RPB)], sb)

        def mk(j, _):
            for k in range(SUB // L):
                sl = pl.ds(k * L, L)
                keyb[j, sl] = pb[j, sl] * N + sb[j, sl]
            return 0

        lax.fori_loop(0, RPB, mk, 0)

        def gc(j, _):
            pltpu.sync_copy(counts_sp.at[keyb.at[j]],
                            valb.at[pl.ds(j * SUB, SUB)])
            return 0

        lax.fori_loop(0, RPB, gc, 0)

        def inv(i, _):
            sl = pl.ds(i * L, L)
            valb[sl] = 1.0 / valb[sl]
            return 0

        lax.fori_loop(0, BLKE // L, inv, 0)

        def mg(j, _):
            for k in range(SUB // L):
                sl = pl.ds(k * L, L)
                keyb[j, sl] = qb[j, sl] * H1 + pb[j, sl]
            return 0

        lax.fori_loop(0, RPB, mg, 0)

        def gr(q, _):
            descs = []
            for k in range(5):
                j = q * 5 + k
                descs.append(pltpu.async_copy(
                    nw2.at[keyb.at[j]], rows.at[pl.ds(j * SUB, SUB)], sem))
            for dsc in descs:
                dsc.wait()
            return 0

        lax.fori_loop(0, RPB // 5, gr, 0)

        def sc_(c, _):
            vch = valb[pl.ds(c * L, L)]
            base = c * L
            for j in range(L):
                v = lax.gather(
                    vch, jnp.full((L, 1), j, jnp.int32), _GDN,
                    slice_sizes=(1,),
                    mode=lax.GatherScatterMode.PROMISE_IN_BOUNDS)
                rows[base + j] = rows[base + j] * v
            return 0

        lax.fori_loop(0, BLKE // L, sc_, 0)

        def sa(j, _):
            pltpu.sync_copy(rows.at[pl.ds(j * SUB, SUB)],
                            out_sp.at[sb.at[j]], add=True)
            return 0

        lax.fori_loop(0, RPB, sa, 0)
        return 0

    lax.fori_loop(0, BBLK, pb_loop, 0)
    plsc.subcore_barrier()

    o0 = sid * OSL
    pltpu.sync_copy(out_sp.at[pl.ds(o0, OSL)], rows.at[pl.ds(0, OSL)])
    pltpu.sync_copy(rows.at[pl.ds(0, OSL)], part.at[cid, pl.ds(o0, OSL)])


@functools.cache
def _sc_kernel():
    mesh = plsc.VectorSubcoreMesh(core_axis_name="c", subcore_axis_name="s")
    return pl.kernel(
        _sc_body,
        out_type=jax.ShapeDtypeStruct((NC, N, H1), jnp.float32),
        mesh=mesh,
        compiler_params=pltpu.CompilerParams(use_tc_tiling_on_sc=False),
        scratch_types=[
            pltpu.VMEM_SHARED((R * N,), jnp.float32),
            pltpu.VMEM_SHARED((N, H1), jnp.float32),
            pltpu.VMEM((RPB, SUB), jnp.int32),
            pltpu.VMEM((RPB, SUB), jnp.int32),
            pltpu.VMEM((RPB, SUB), jnp.int32),
            pltpu.VMEM((RPB, SUB), jnp.int32),
            pltpu.VMEM((BLKE,), jnp.float32),
            pltpu.VMEM((SUB,), jnp.float32),
            pltpu.VMEM((BLKE, H1), jnp.float32),
            pltpu.SemaphoreType.DMA,
        ],
    )


def kernel(nodes, triples, weights, bias):
    s = triples[:, 0].reshape(EROWS, SUB)
    p = triples[:, 1].reshape(EROWS, SUB)
    o = triples[:, 2].reshape(EROWS, SUB)
    w_all = jnp.transpose(weights, (1, 0, 2)).reshape(H0, R * H1)
    nw2 = _mm(nodes, w_all).reshape(N * R, H1)
    part = _sc_kernel()(s, p, o, nw2)
    bias_t = jnp.tile(bias, H0 // H1).reshape(1, 128)
    out2 = _comb(part.reshape(NC, (N * H1) // 128, 128), bias_t)
    return out2.reshape(N, H1)
